# baseline (speedup 1.0000x reference)
import functools

import jax
import jax.numpy as jnp
from jax import lax
from jax.experimental import pallas as pl
from jax.experimental.pallas import tpu as pltpu
from jax.experimental.pallas import tpu_sc as plsc

N_NODES = 10000
N_EDGES = 320000
IN_DIM = 128
HID_DIM = 64
OUT_DIM = 32

NC = 2
NS = 16
CHUNK = 128
CHUNKS_PER_TILE = 80
E_PAD = NC * NS * CHUNKS_PER_TILE * CHUNK
ROWS_PER_TILE = N_NODES // NS

_MESH = dict(core_axis_name="c", subcore_axis_name="s", num_cores=NC,
             num_subcores=NS)
_SC_PARAMS = pltpu.CompilerParams(needs_layout_passes=False,
                                  use_tc_tiling_on_sc=False)


def _deg_body(dst_hbm, ew_hbm, degp_hbm, dst_v, ew_v, degp_v):
    c = lax.axis_index("c")
    s = lax.axis_index("s")
    wid = c * NS + s
    pltpu.sync_copy(dst_hbm.at[wid], dst_v)
    pltpu.sync_copy(ew_hbm.at[wid], ew_v)

    zi16 = jnp.zeros((16,), jnp.int32)
    lane = lax.iota(jnp.int32, 16)

    def zero(i, _):
        plsc.store_scatter(degp_v, [i * 16 + lane, zi16],
                           jnp.zeros((16,), jnp.float32))
        return 0

    lax.fori_loop(0, N_NODES // 16, zero, 0)

    def chunk(j, _):
        for k in range(CHUNK // 16):
            idx = dst_v[j, pl.ds(k * 16, 16)]
            val = ew_v[j, pl.ds(k * 16, 16)]
            plsc.addupdate_scatter(degp_v, [idx, zi16], val)
        return 0

    lax.fori_loop(0, CHUNKS_PER_TILE, chunk, 0)
    pltpu.sync_copy(degp_v, degp_hbm.at[wid])


@jax.jit
def _deg_call(dst_p, ew_p):
    return pl.kernel(
        _deg_body,
        out_type=jax.ShapeDtypeStruct((NC * NS, N_NODES, 1), jnp.float32),
        mesh=plsc.VectorSubcoreMesh(**_MESH),
        scratch_types=[
            pltpu.VMEM((CHUNKS_PER_TILE, CHUNK), jnp.int32),
            pltpu.VMEM((CHUNKS_PER_TILE, CHUNK), jnp.float32),
            pltpu.VMEM((N_NODES, 1), jnp.float32),
        ],
        compiler_params=_SC_PARAMS,
    )(dst_p, ew_p)


def _agg_body(d, g_hbm, src_hbm, dst_hbm, ew_hbm, aggp_hbm,
              src_v, dst_v, ew_v, rows_a, rows_b, z_v, acc_sh,
              gs_a, gs_b, ss_a, ss_b):
    nd = d // 16
    c = lax.axis_index("c")
    s = lax.axis_index("s")
    wid = c * NS + s
    pltpu.sync_copy(src_hbm.at[wid], src_v)
    pltpu.sync_copy(dst_hbm.at[wid], dst_v)
    pltpu.sync_copy(ew_hbm.at[wid], ew_v)

    def zero(i, _):
        for k in range(nd):
            z_v[i, pl.ds(k * 16, 16)] = jnp.zeros((16,), jnp.float32)
        return 0

    lax.fori_loop(0, ROWS_PER_TILE, zero, 0)
    pltpu.sync_copy(z_v, acc_sh.at[pl.ds(s * ROWS_PER_TILE, ROWS_PER_TILE)])
    plsc.subcore_barrier()

    def scale(j, buf):
        def body(i, _):
            sc = plsc.load_gather(
                ew_v, [jnp.full((16,), j, jnp.int32), jnp.full((16,), i, jnp.int32)])
            for k in range(nd):
                buf[i, pl.ds(k * 16, 16)] = buf[i, pl.ds(k * 16, 16)] * sc
            return 0

        lax.fori_loop(0, CHUNK, body, 0)

    def issue_gather(j, buf, sem):
        pltpu.async_copy(g_hbm.at[src_v.at[j]], buf, sem)

    def wait_gather(buf, sem):
        pltpu.make_async_copy(g_hbm.at[src_v.at[0]], buf, sem).wait()

    def issue_scatter(j, buf, sem):
        pltpu.async_copy(buf, acc_sh.at[dst_v.at[j]], sem, add=True)

    def wait_scatter(buf, sem):
        pltpu.make_async_copy(buf, acc_sh.at[dst_v.at[0]], sem).wait()

    def pair(k, first, last):
        j0 = 2 * k
        j1 = j0 + 1
        wait_gather(rows_a, gs_a)
        if not first:
            wait_scatter(rows_b, ss_b)
        issue_gather(j1, rows_b, gs_b)
        scale(j0, rows_a)
        issue_scatter(j0, rows_a, ss_a)
        wait_gather(rows_b, gs_b)
        scale(j1, rows_b)
        wait_scatter(rows_a, ss_a)
        if not last:
            issue_gather(j0 + 2, rows_a, gs_a)
        issue_scatter(j1, rows_b, ss_b)

    npairs = CHUNKS_PER_TILE // 2
    issue_gather(0, rows_a, gs_a)
    pair(0, True, False)

    def mid(k, _):
        pair(k, False, False)
        return 0

    lax.fori_loop(1, npairs - 1, mid, 0)
    pair(npairs - 1, False, True)
    wait_scatter(rows_b, ss_b)
    plsc.subcore_barrier()
    pltpu.sync_copy(
        acc_sh.at[pl.ds(s * ROWS_PER_TILE, ROWS_PER_TILE)],
        aggp_hbm.at[c, pl.ds(s * ROWS_PER_TILE, ROWS_PER_TILE)])


@functools.partial(jax.jit, static_argnums=0)
def _agg_call(d, g, src_p, dst_p, ew_p):
    return pl.kernel(
        functools.partial(_agg_body, d),
        out_type=jax.ShapeDtypeStruct((NC, N_NODES, d), jnp.float32),
        mesh=plsc.VectorSubcoreMesh(**_MESH),
        scratch_types=[
            pltpu.VMEM((CHUNKS_PER_TILE, CHUNK), jnp.int32),
            pltpu.VMEM((CHUNKS_PER_TILE, CHUNK), jnp.int32),
            pltpu.VMEM((CHUNKS_PER_TILE, CHUNK), jnp.float32),
            pltpu.VMEM((CHUNK, d), jnp.float32),
            pltpu.VMEM((CHUNK, d), jnp.float32),
            pltpu.VMEM((ROWS_PER_TILE, d), jnp.float32),
            pltpu.VMEM_SHARED((N_NODES, d), jnp.float32),
            pltpu.SemaphoreType.DMA,
            pltpu.SemaphoreType.DMA,
            pltpu.SemaphoreType.DMA,
            pltpu.SemaphoreType.DMA,
        ],
        compiler_params=_SC_PARAMS,
    )(g, src_p, dst_p, ew_p)


_TCR = 1000


def _tc1_body(degp_ref, x_ref, w1t_ref, dis_ref, g_ref):
    deg = 1.0 + jnp.sum(degp_ref[...], axis=1, keepdims=True)
    pos = deg > 0
    dis = jnp.where(pos, lax.rsqrt(jnp.where(pos, deg, 1.0)), 0.0)
    h = jnp.dot(x_ref[...], w1t_ref[...], preferred_element_type=jnp.float32)
    dis_ref[...] = dis
    g_ref[...] = dis * h


def _tc2_body(aggp_ref, g1_ref, dis_ref, b1_ref, w2t_ref, g2_ref):
    agg = aggp_ref[0] + aggp_ref[1] + g1_ref[...]
    out1 = jnp.maximum(dis_ref[...] * agg + b1_ref[...], 0.0)
    g2_ref[...] = dis_ref[...] * jnp.dot(
        out1, w2t_ref[...], preferred_element_type=jnp.float32)


def _tc3_body(aggp_ref, g2_ref, dis_ref, b2_ref, out_ref):
    agg = aggp_ref[0] + aggp_ref[1] + g2_ref[...]
    out_ref[...] = dis_ref[...] * agg + b2_ref[...]


@jax.jit
def _tc1_call(deg2, x, w1t):
    grid = (N_NODES // _TCR,)
    return pl.pallas_call(
        _tc1_body,
        grid=grid,
        in_specs=[
            pl.BlockSpec((_TCR, NC * NS), lambda i: (i, 0)),
            pl.BlockSpec((_TCR, IN_DIM), lambda i: (i, 0)),
            pl.BlockSpec((IN_DIM, HID_DIM), lambda i: (0, 0)),
        ],
        out_specs=[
            pl.BlockSpec((_TCR, 1), lambda i: (i, 0)),
            pl.BlockSpec((_TCR, HID_DIM), lambda i: (i, 0)),
        ],
        out_shape=[
            jax.ShapeDtypeStruct((N_NODES, 1), jnp.float32),
            jax.ShapeDtypeStruct((N_NODES, HID_DIM), jnp.float32),
        ],
    )(deg2, x, w1t)


@jax.jit
def _tc2_call(aggp1, g1, dis, b1r, w2t):
    grid = (N_NODES // _TCR,)
    return pl.pallas_call(
        _tc2_body,
        grid=grid,
        in_specs=[
            pl.BlockSpec((NC, _TCR, HID_DIM), lambda i: (0, i, 0)),
            pl.BlockSpec((_TCR, HID_DIM), lambda i: (i, 0)),
            pl.BlockSpec((_TCR, 1), lambda i: (i, 0)),
            pl.BlockSpec((1, HID_DIM), lambda i: (0, 0)),
            pl.BlockSpec((HID_DIM, OUT_DIM), lambda i: (0, 0)),
        ],
        out_specs=pl.BlockSpec((_TCR, OUT_DIM), lambda i: (i, 0)),
        out_shape=jax.ShapeDtypeStruct((N_NODES, OUT_DIM), jnp.float32),
    )(aggp1, g1, dis, b1r, w2t)


@jax.jit
def _tc3_call(aggp2, g2, dis, b2r):
    grid = (N_NODES // _TCR,)
    return pl.pallas_call(
        _tc3_body,
        grid=grid,
        in_specs=[
            pl.BlockSpec((NC, _TCR, OUT_DIM), lambda i: (0, i, 0)),
            pl.BlockSpec((_TCR, OUT_DIM), lambda i: (i, 0)),
            pl.BlockSpec((_TCR, 1), lambda i: (i, 0)),
            pl.BlockSpec((1, OUT_DIM), lambda i: (0, 0)),
        ],
        out_specs=pl.BlockSpec((_TCR, OUT_DIM), lambda i: (i, 0)),
        out_shape=jax.ShapeDtypeStruct((N_NODES, OUT_DIM), jnp.float32),
    )(aggp2, g2, dis, b2r)


def kernel(x, edge_index, edge_weight, W1, b1, W2, b2):
    src = edge_index[0]
    dst = edge_index[1]
    pad = E_PAD - N_EDGES
    zi = jnp.zeros((pad,), dst.dtype)
    shard = (NC * NS, CHUNKS_PER_TILE, CHUNK)
    src_p = jnp.concatenate([src, zi]).reshape(shard)
    dst_p = jnp.concatenate([dst, zi]).reshape(shard)
    ew_p = jnp.concatenate(
        [edge_weight, jnp.zeros((pad,), edge_weight.dtype)]).reshape(shard)

    degp = _deg_call(dst_p, ew_p)
    deg2 = degp[..., 0].T
    dis, g1 = _tc1_call(deg2, x, W1.T)
    aggp1 = _agg_call(HID_DIM, g1, src_p, dst_p, ew_p)
    g2 = _tc2_call(aggp1, g1, dis, b1.reshape(1, -1), W2.T)
    aggp2 = _agg_call(OUT_DIM, g2, src_p, dst_p, ew_p)
    return _tc3_call(aggp2, g2, dis, b2.reshape(1, -1))


# speedup vs baseline: 1.4528x; 1.4528x over previous
import functools

import jax
import jax.numpy as jnp
from jax import lax
from jax.experimental import pallas as pl
from jax.experimental.pallas import tpu as pltpu
from jax.experimental.pallas import tpu_sc as plsc

N_NODES = 10000
N_EDGES = 320000
IN_DIM = 128
HID_DIM = 64
OUT_DIM = 32

NC = 2
NS = 16
CHUNK = 100
CHUNKS_PER_TILE = 100
EDGES_PER_TILE = CHUNKS_PER_TILE * CHUNK
ROWS_PER_TILE = N_NODES // NS

_MESH = dict(core_axis_name="c", subcore_axis_name="s", num_cores=NC,
             num_subcores=NS)
_SC_PARAMS = pltpu.CompilerParams(needs_layout_passes=False,
                                  use_tc_tiling_on_sc=False)


def _deg_body(dst_hbm, ew_hbm, degp_hbm, dst_v, ew_v, degp_v):
    c = lax.axis_index("c")
    s = lax.axis_index("s")
    wid = c * NS + s
    pltpu.sync_copy(dst_hbm.at[wid], dst_v)
    pltpu.sync_copy(ew_hbm.at[wid], ew_v)

    zi16 = jnp.zeros((16,), jnp.int32)
    lane = lax.iota(jnp.int32, 16)

    def zero(i, _):
        plsc.store_scatter(degp_v, [i * 16 + lane, zi16],
                           jnp.zeros((16,), jnp.float32))
        return 0

    lax.fori_loop(0, N_NODES // 16, zero, 0)

    def chunk(j, _):
        idx = dst_v[pl.ds(j * 16, 16)]
        val = ew_v[pl.ds(j * 16, 16)]
        plsc.addupdate_scatter(degp_v, [idx, zi16], val)
        return 0

    lax.fori_loop(0, EDGES_PER_TILE // 16, chunk, 0)
    pltpu.sync_copy(degp_v, degp_hbm.at[wid])


@jax.jit
def _deg_call(dst_p, ew_p):
    return pl.kernel(
        _deg_body,
        out_type=jax.ShapeDtypeStruct((NC * NS, N_NODES, 1), jnp.float32),
        mesh=plsc.VectorSubcoreMesh(**_MESH),
        scratch_types=[
            pltpu.VMEM((EDGES_PER_TILE,), jnp.int32),
            pltpu.VMEM((EDGES_PER_TILE,), jnp.float32),
            pltpu.VMEM((N_NODES, 1), jnp.float32),
        ],
        compiler_params=_SC_PARAMS,
    )(dst_p, ew_p)


def _agg_body(d, g_hbm, src_hbm, dst_hbm, ew_hbm, aggp_hbm,
              src_v, dst_v, ew_v, rows_a, rows_b, z_v, acc_sh,
              gs_a, gs_b, ss_a, ss_b):
    nd = d // 16
    c = lax.axis_index("c")
    s = lax.axis_index("s")
    wid = c * NS + s
    pltpu.sync_copy(src_hbm.at[wid], src_v)
    pltpu.sync_copy(dst_hbm.at[wid], dst_v)
    pltpu.sync_copy(ew_hbm.at[wid], ew_v)

    def zero(i, _):
        for k in range(nd):
            z_v[i, pl.ds(k * 16, 16)] = jnp.zeros((16,), jnp.float32)
        return 0

    lax.fori_loop(0, ROWS_PER_TILE, zero, 0)
    pltpu.sync_copy(z_v, acc_sh.at[pl.ds(s * ROWS_PER_TILE, ROWS_PER_TILE)])
    plsc.subcore_barrier()

    def scale(j, buf):
        def body(i, _):
            sc = plsc.load_gather(
                ew_v, [jnp.full((16,), j, jnp.int32), jnp.full((16,), i, jnp.int32)])
            for k in range(nd):
                buf[i, pl.ds(k * 16, 16)] = buf[i, pl.ds(k * 16, 16)] * sc
            return 0

        lax.fori_loop(0, CHUNK, body, 0)

    def issue_gather(j, buf, sem):
        pltpu.async_copy(g_hbm.at[src_v.at[j]], buf, sem)

    def wait_gather(buf, sem):
        pltpu.make_async_copy(g_hbm.at[src_v.at[0]], buf, sem).wait()

    def issue_scatter(j, buf, sem):
        pltpu.async_copy(buf, acc_sh.at[dst_v.at[j]], sem, add=True)

    def wait_scatter(buf, sem):
        pltpu.make_async_copy(buf, acc_sh.at[dst_v.at[0]], sem).wait()

    def pair(k, first, last):
        j0 = 2 * k
        j1 = j0 + 1
        wait_gather(rows_a, gs_a)
        if not first:
            wait_scatter(rows_b, ss_b)
        issue_gather(j1, rows_b, gs_b)
        scale(j0, rows_a)
        issue_scatter(j0, rows_a, ss_a)
        wait_gather(rows_b, gs_b)
        scale(j1, rows_b)
        wait_scatter(rows_a, ss_a)
        if not last:
            issue_gather(j0 + 2, rows_a, gs_a)
        issue_scatter(j1, rows_b, ss_b)

    npairs = CHUNKS_PER_TILE // 2
    issue_gather(0, rows_a, gs_a)
    pair(0, True, False)

    def mid(k, _):
        pair(k, False, False)
        return 0

    lax.fori_loop(1, npairs - 1, mid, 0)
    pair(npairs - 1, False, True)
    wait_scatter(rows_b, ss_b)
    plsc.subcore_barrier()
    pltpu.sync_copy(
        acc_sh.at[pl.ds(s * ROWS_PER_TILE, ROWS_PER_TILE)],
        aggp_hbm.at[c, pl.ds(s * ROWS_PER_TILE, ROWS_PER_TILE)])


@functools.partial(jax.jit, static_argnums=0)
def _agg_call(d, g, src_p, dst_p, ew_p):
    return pl.kernel(
        functools.partial(_agg_body, d),
        out_type=jax.ShapeDtypeStruct((NC, N_NODES, d), jnp.float32),
        mesh=plsc.VectorSubcoreMesh(**_MESH),
        scratch_types=[
            pltpu.VMEM((CHUNKS_PER_TILE, CHUNK), jnp.int32),
            pltpu.VMEM((CHUNKS_PER_TILE, CHUNK), jnp.int32),
            pltpu.VMEM((CHUNKS_PER_TILE, CHUNK), jnp.float32),
            pltpu.VMEM((CHUNK, d), jnp.float32),
            pltpu.VMEM((CHUNK, d), jnp.float32),
            pltpu.VMEM((ROWS_PER_TILE, d), jnp.float32),
            pltpu.VMEM_SHARED((N_NODES, d), jnp.float32),
            pltpu.SemaphoreType.DMA,
            pltpu.SemaphoreType.DMA,
            pltpu.SemaphoreType.DMA,
            pltpu.SemaphoreType.DMA,
        ],
        compiler_params=_SC_PARAMS,
    )(g, src_p, dst_p, ew_p)


_TCR = 1000


def _dis_body(degp_ref, dis_ref):
    degp_t = jnp.transpose(degp_ref[...])
    deg = 1.0 + jnp.sum(degp_t, axis=1, keepdims=True)
    pos = deg > 0
    dis_ref[...] = jnp.where(pos, lax.rsqrt(jnp.where(pos, deg, 1.0)), 0.0)


def _tc1_body(dis_ref, x_ref, w1t_ref, g_ref):
    h = jnp.dot(x_ref[...], w1t_ref[...], preferred_element_type=jnp.float32)
    g_ref[...] = dis_ref[...] * h


def _tc2_body(aggp_ref, g1_ref, dis_ref, b1_ref, w2t_ref, g2_ref):
    agg = aggp_ref[0] + aggp_ref[1] + g1_ref[...]
    out1 = jnp.maximum(dis_ref[...] * agg + b1_ref[...], 0.0)
    g2_ref[...] = dis_ref[...] * jnp.dot(
        out1, w2t_ref[...], preferred_element_type=jnp.float32)


def _tc3_body(aggp_ref, g2_ref, dis_ref, b2_ref, out_ref):
    agg = aggp_ref[0] + aggp_ref[1] + g2_ref[...]
    out_ref[...] = dis_ref[...] * agg + b2_ref[...]


@jax.jit
def _dis_call(degp2):
    return pl.pallas_call(
        _dis_body,
        out_shape=jax.ShapeDtypeStruct((N_NODES, 1), jnp.float32),
    )(degp2)


@jax.jit
def _tc1_call(dis, x, w1t):
    return pl.pallas_call(
        _tc1_body,
        out_shape=jax.ShapeDtypeStruct((N_NODES, HID_DIM), jnp.float32),
    )(dis, x, w1t)


@jax.jit
def _tc2_call(aggp1, g1, dis, b1r, w2t):
    return pl.pallas_call(
        _tc2_body,
        out_shape=jax.ShapeDtypeStruct((N_NODES, OUT_DIM), jnp.float32),
    )(aggp1, g1, dis, b1r, w2t)


@jax.jit
def _tc3_call(aggp2, g2, dis, b2r):
    return pl.pallas_call(
        _tc3_body,
        out_shape=jax.ShapeDtypeStruct((N_NODES, OUT_DIM), jnp.float32),
    )(aggp2, g2, dis, b2r)


def kernel(x, edge_index, edge_weight, W1, b1, W2, b2):
    src = edge_index[0]
    dst = edge_index[1]
    shard = (NC * NS, CHUNKS_PER_TILE, CHUNK)
    flat = (NC * NS, EDGES_PER_TILE)
    src_p = src.reshape(shard)
    dst_p = dst.reshape(shard)
    ew_p = edge_weight.reshape(shard)

    degp = _deg_call(dst.reshape(flat), edge_weight.reshape(flat))
    dis = _dis_call(degp.reshape(NC * NS, N_NODES))
    g1 = _tc1_call(dis, x, W1.T)
    aggp1 = _agg_call(HID_DIM, g1, src_p, dst_p, ew_p)
    g2 = _tc2_call(aggp1, g1, dis, b1.reshape(1, -1), W2.T)
    aggp2 = _agg_call(OUT_DIM, g2, src_p, dst_p, ew_p)
    return _tc3_call(aggp2, g2, dis, b2.reshape(1, -1))


# speedup vs baseline: 2.1062x; 1.4498x over previous
import functools

import jax
import jax.numpy as jnp
from jax import lax
from jax.experimental import pallas as pl
from jax.experimental.pallas import tpu as pltpu
from jax.experimental.pallas import tpu_sc as plsc

N_NODES = 10000
N_EDGES = 320000
IN_DIM = 128
HID_DIM = 64
OUT_DIM = 32

NC = 2
NS = 16
CHUNK = 100
CHUNKS_PER_TILE = 100
EDGES_PER_TILE = CHUNKS_PER_TILE * CHUNK
ROWS_PER_TILE = N_NODES // NS

_MESH = dict(core_axis_name="c", subcore_axis_name="s", num_cores=NC,
             num_subcores=NS)
_SC_PARAMS = pltpu.CompilerParams(needs_layout_passes=False,
                                  use_tc_tiling_on_sc=False)


def _deg_body(dst_hbm, ew_hbm, degp_hbm, dst_v, ew_v, degp_v):
    c = lax.axis_index("c")
    s = lax.axis_index("s")
    wid = c * NS + s
    pltpu.sync_copy(dst_hbm.at[wid], dst_v)
    pltpu.sync_copy(ew_hbm.at[wid], ew_v)

    def zero(i, _):
        degp_v[pl.ds(i * 16, 16)] = jnp.zeros((16,), jnp.float32)
        return 0

    lax.fori_loop(0, N_NODES // 16, zero, 0)

    def chunk(j, _):
        idx = dst_v[pl.ds(j * 16, 16)]
        val = ew_v[pl.ds(j * 16, 16)]
        plsc.addupdate_scatter(degp_v, [idx], val)
        return 0

    lax.fori_loop(0, EDGES_PER_TILE // 16, chunk, 0)
    pltpu.sync_copy(degp_v, degp_hbm.at[wid])


@jax.jit
def _deg_call(dst_p, ew_p):
    return pl.kernel(
        _deg_body,
        out_type=jax.ShapeDtypeStruct((NC * NS, N_NODES), jnp.float32),
        mesh=plsc.VectorSubcoreMesh(**_MESH),
        scratch_types=[
            pltpu.VMEM((EDGES_PER_TILE,), jnp.int32),
            pltpu.VMEM((EDGES_PER_TILE,), jnp.float32),
            pltpu.VMEM((N_NODES,), jnp.float32),
        ],
        compiler_params=_SC_PARAMS,
    )(dst_p, ew_p)


def _agg_body(d, g_hbm, src_hbm, dst_hbm, ew_hbm, aggp_hbm,
              src_v, dst_v, ew_v, rows_a, rows_b, z_v, acc_sh,
              gs_a, gs_b, ss_a, ss_b):
    nd = d // 16
    c = lax.axis_index("c")
    s = lax.axis_index("s")
    wid = c * NS + s
    pltpu.sync_copy(src_hbm.at[wid], src_v)
    pltpu.sync_copy(dst_hbm.at[wid], dst_v)
    pltpu.sync_copy(ew_hbm.at[wid], ew_v)

    def zero(i, _):
        for k in range(nd):
            z_v[i, pl.ds(k * 16, 16)] = jnp.zeros((16,), jnp.float32)
        return 0

    lax.fori_loop(0, ROWS_PER_TILE, zero, 0)
    pltpu.sync_copy(z_v, acc_sh.at[pl.ds(s * ROWS_PER_TILE, ROWS_PER_TILE)])
    plsc.subcore_barrier()

    def scale(j, buf):
        def body(i, _):
            sc = plsc.load_gather(
                ew_v, [jnp.full((16,), j, jnp.int32), jnp.full((16,), i, jnp.int32)])
            for k in range(nd):
                buf[i, pl.ds(k * 16, 16)] = buf[i, pl.ds(k * 16, 16)] * sc
            return 0

        lax.fori_loop(0, CHUNK, body, 0)

    def issue_gather(j, buf, sem):
        pltpu.async_copy(g_hbm.at[src_v.at[j]], buf, sem)

    def wait_gather(buf, sem):
        pltpu.make_async_copy(g_hbm.at[src_v.at[0]], buf, sem).wait()

    def issue_scatter(j, buf, sem):
        pltpu.async_copy(buf, acc_sh.at[dst_v.at[j]], sem, add=True)

    def wait_scatter(buf, sem):
        pltpu.make_async_copy(buf, acc_sh.at[dst_v.at[0]], sem).wait()

    def pair(k, first, last):
        j0 = 2 * k
        j1 = j0 + 1
        wait_gather(rows_a, gs_a)
        if not first:
            wait_scatter(rows_b, ss_b)
        issue_gather(j1, rows_b, gs_b)
        scale(j0, rows_a)
        issue_scatter(j0, rows_a, ss_a)
        wait_gather(rows_b, gs_b)
        scale(j1, rows_b)
        wait_scatter(rows_a, ss_a)
        if not last:
            issue_gather(j0 + 2, rows_a, gs_a)
        issue_scatter(j1, rows_b, ss_b)

    npairs = CHUNKS_PER_TILE // 2
    issue_gather(0, rows_a, gs_a)
    pair(0, True, False)

    def mid(k, _):
        pair(k, False, False)
        return 0

    lax.fori_loop(1, npairs - 1, mid, 0)
    pair(npairs - 1, False, True)
    wait_scatter(rows_b, ss_b)
    plsc.subcore_barrier()
    pltpu.sync_copy(
        acc_sh.at[pl.ds(s * ROWS_PER_TILE, ROWS_PER_TILE)],
        aggp_hbm.at[c, pl.ds(s * ROWS_PER_TILE, ROWS_PER_TILE)])


@functools.partial(jax.jit, static_argnums=0)
def _agg_call(d, g, src_p, dst_p, ew_p):
    return pl.kernel(
        functools.partial(_agg_body, d),
        out_type=jax.ShapeDtypeStruct((NC, N_NODES, d), jnp.float32),
        mesh=plsc.VectorSubcoreMesh(**_MESH),
        scratch_types=[
            pltpu.VMEM((CHUNKS_PER_TILE, CHUNK), jnp.int32),
            pltpu.VMEM((CHUNKS_PER_TILE, CHUNK), jnp.int32),
            pltpu.VMEM((CHUNKS_PER_TILE, CHUNK), jnp.float32),
            pltpu.VMEM((CHUNK, d), jnp.float32),
            pltpu.VMEM((CHUNK, d), jnp.float32),
            pltpu.VMEM((ROWS_PER_TILE, d), jnp.float32),
            pltpu.VMEM_SHARED((N_NODES, d), jnp.float32),
            pltpu.SemaphoreType.DMA,
            pltpu.SemaphoreType.DMA,
            pltpu.SemaphoreType.DMA,
            pltpu.SemaphoreType.DMA,
        ],
        compiler_params=_SC_PARAMS,
    )(g, src_p, dst_p, ew_p)


_TCR = 1000


def _tc1_body(degp_ref, x_ref, w1t_ref, dis_ref, g_ref):
    degp_t = jnp.transpose(degp_ref[...])
    deg = 1.0 + jnp.sum(degp_t, axis=1, keepdims=True)
    pos = deg > 0
    dis = jnp.where(pos, lax.rsqrt(jnp.where(pos, deg, 1.0)), 0.0)
    h = jnp.dot(x_ref[...], w1t_ref[...], preferred_element_type=jnp.float32)
    dis_ref[...] = dis
    g_ref[...] = dis * h


def _tc2_body(aggp_ref, g1_ref, dis_ref, b1_ref, w2t_ref, g2_ref):
    agg = aggp_ref[0] + aggp_ref[1] + g1_ref[...]
    out1 = jnp.maximum(dis_ref[...] * agg + b1_ref[...], 0.0)
    g2_ref[...] = dis_ref[...] * jnp.dot(
        out1, w2t_ref[...], preferred_element_type=jnp.float32)


def _tc3_body(aggp_ref, g2_ref, dis_ref, b2_ref, out_ref):
    agg = aggp_ref[0] + aggp_ref[1] + g2_ref[...]
    out_ref[...] = dis_ref[...] * agg + b2_ref[...]


@jax.jit
def _tc1_call(degp, x, w1t):
    return pl.pallas_call(
        _tc1_body,
        out_shape=[
            jax.ShapeDtypeStruct((N_NODES, 1), jnp.float32),
            jax.ShapeDtypeStruct((N_NODES, HID_DIM), jnp.float32),
        ],
    )(degp, x, w1t)


@jax.jit
def _tc2_call(aggp1, g1, dis, b1r, w2t):
    return pl.pallas_call(
        _tc2_body,
        out_shape=jax.ShapeDtypeStruct((N_NODES, OUT_DIM), jnp.float32),
    )(aggp1, g1, dis, b1r, w2t)


@jax.jit
def _tc3_call(aggp2, g2, dis, b2r):
    return pl.pallas_call(
        _tc3_body,
        out_shape=jax.ShapeDtypeStruct((N_NODES, OUT_DIM), jnp.float32),
    )(aggp2, g2, dis, b2r)


def kernel(x, edge_index, edge_weight, W1, b1, W2, b2):
    src = edge_index[0]
    dst = edge_index[1]
    shard = (NC * NS, CHUNKS_PER_TILE, CHUNK)
    flat = (NC * NS, EDGES_PER_TILE)
    src_p = src.reshape(shard)
    dst_p = dst.reshape(shard)
    ew_p = edge_weight.reshape(shard)

    degp = _deg_call(dst.reshape(flat), edge_weight.reshape(flat))
    dis, g1 = _tc1_call(degp, x, W1.T)
    aggp1 = _agg_call(HID_DIM, g1, src_p, dst_p, ew_p)
    g2 = _tc2_call(aggp1, g1, dis, b1.reshape(1, -1), W2.T)
    aggp2 = _agg_call(OUT_DIM, g2, src_p, dst_p, ew_p)
    return _tc3_call(aggp2, g2, dis, b2.reshape(1, -1))


# speedup vs baseline: 2.2861x; 1.0855x over previous
import functools

import jax
import jax.numpy as jnp
from jax import lax
from jax.experimental import pallas as pl
from jax.experimental.pallas import tpu as pltpu
from jax.experimental.pallas import tpu_sc as plsc

N_NODES = 10000
N_EDGES = 320000
IN_DIM = 128
HID_DIM = 64
OUT_DIM = 32

NC = 2
NS = 16
CHUNK = 100
CHUNKS_PER_TILE = 100
EDGES_PER_TILE = CHUNKS_PER_TILE * CHUNK
ROWS_PER_TILE = N_NODES // NS

_MESH = dict(core_axis_name="c", subcore_axis_name="s", num_cores=NC,
             num_subcores=NS)
_SC_PARAMS = pltpu.CompilerParams(needs_layout_passes=False,
                                  use_tc_tiling_on_sc=False)


def _deg_body(dst_hbm, ew_hbm, degp_hbm, dst_v, ew_v, degp_v):
    c = lax.axis_index("c")
    s = lax.axis_index("s")
    wid = c * NS + s
    pltpu.sync_copy(dst_hbm.at[wid], dst_v)
    pltpu.sync_copy(ew_hbm.at[wid], ew_v)

    def zero(i, _):
        degp_v[pl.ds(i * 16, 16)] = jnp.zeros((16,), jnp.float32)
        return 0

    lax.fori_loop(0, N_NODES // 16, zero, 0)

    def chunk(j, _):
        idx = dst_v[pl.ds(j * 16, 16)]
        val = ew_v[pl.ds(j * 16, 16)]
        plsc.addupdate_scatter(degp_v, [idx], val)
        return 0

    lax.fori_loop(0, EDGES_PER_TILE // 16, chunk, 0)
    pltpu.sync_copy(degp_v, degp_hbm.at[wid])


@jax.jit
def _deg_call(dst_p, ew_p):
    return pl.kernel(
        _deg_body,
        out_type=jax.ShapeDtypeStruct((NC * NS, N_NODES), jnp.float32),
        mesh=plsc.VectorSubcoreMesh(**_MESH),
        scratch_types=[
            pltpu.VMEM((EDGES_PER_TILE,), jnp.int32),
            pltpu.VMEM((EDGES_PER_TILE,), jnp.float32),
            pltpu.VMEM((N_NODES,), jnp.float32),
        ],
        compiler_params=_SC_PARAMS,
    )(dst_p, ew_p)


def _agg_body(d, stage_g, g_hbm, src_hbm, dst_hbm, ew_hbm, aggp_hbm,
              src_v, dst_v, ew_v, rows_a, rows_b, z_v, acc_sh,
              gs_a, gs_b, ss_a, ss_b, *maybe_g_sh):
    nd = d // 16
    c = lax.axis_index("c")
    s = lax.axis_index("s")
    wid = c * NS + s
    pltpu.sync_copy(src_hbm.at[wid], src_v)
    pltpu.sync_copy(dst_hbm.at[wid], dst_v)
    pltpu.sync_copy(ew_hbm.at[wid], ew_v)
    if stage_g:
        g_src = maybe_g_sh[0]
        row0 = s * ROWS_PER_TILE
        pltpu.sync_copy(g_hbm.at[pl.ds(row0, ROWS_PER_TILE)],
                        g_src.at[pl.ds(row0, ROWS_PER_TILE)])
    else:
        g_src = g_hbm

    def zero(i, _):
        for k in range(nd):
            z_v[i, pl.ds(k * 16, 16)] = jnp.zeros((16,), jnp.float32)
        return 0

    lax.fori_loop(0, ROWS_PER_TILE, zero, 0)
    pltpu.sync_copy(z_v, acc_sh.at[pl.ds(s * ROWS_PER_TILE, ROWS_PER_TILE)])
    plsc.subcore_barrier()

    def scale(j, buf):
        def body(i, _):
            sc = plsc.load_gather(
                ew_v, [jnp.full((16,), j, jnp.int32), jnp.full((16,), i, jnp.int32)])
            for k in range(nd):
                buf[i, pl.ds(k * 16, 16)] = buf[i, pl.ds(k * 16, 16)] * sc
            return 0

        lax.fori_loop(0, CHUNK, body, 0)

    def issue_gather(j, buf, sem):
        pltpu.async_copy(g_src.at[src_v.at[j]], buf, sem)

    def wait_gather(buf, sem):
        pltpu.make_async_copy(g_src.at[src_v.at[0]], buf, sem).wait()

    def issue_scatter(j, buf, sem):
        pltpu.async_copy(buf, acc_sh.at[dst_v.at[j]], sem, add=True)

    def wait_scatter(buf, sem):
        pltpu.make_async_copy(buf, acc_sh.at[dst_v.at[0]], sem).wait()

    def pair(k, first, last):
        j0 = 2 * k
        j1 = j0 + 1
        wait_gather(rows_a, gs_a)
        if not first:
            wait_scatter(rows_b, ss_b)
        issue_gather(j1, rows_b, gs_b)
        scale(j0, rows_a)
        issue_scatter(j0, rows_a, ss_a)
        wait_gather(rows_b, gs_b)
        scale(j1, rows_b)
        wait_scatter(rows_a, ss_a)
        if not last:
            issue_gather(j0 + 2, rows_a, gs_a)
        issue_scatter(j1, rows_b, ss_b)

    npairs = CHUNKS_PER_TILE // 2
    issue_gather(0, rows_a, gs_a)
    pair(0, True, False)

    def mid(k, _):
        pair(k, False, False)
        return 0

    lax.fori_loop(1, npairs - 1, mid, 0)
    pair(npairs - 1, False, True)
    wait_scatter(rows_b, ss_b)
    plsc.subcore_barrier()
    pltpu.sync_copy(
        acc_sh.at[pl.ds(s * ROWS_PER_TILE, ROWS_PER_TILE)],
        aggp_hbm.at[c, pl.ds(s * ROWS_PER_TILE, ROWS_PER_TILE)])


@functools.partial(jax.jit, static_argnums=(0, 1))
def _agg_call(d, stage_g, g, src_p, dst_p, ew_p):
    scratch = [
        pltpu.VMEM((CHUNKS_PER_TILE, CHUNK), jnp.int32),
        pltpu.VMEM((CHUNKS_PER_TILE, CHUNK), jnp.int32),
        pltpu.VMEM((CHUNKS_PER_TILE, CHUNK), jnp.float32),
        pltpu.VMEM((CHUNK, d), jnp.float32),
        pltpu.VMEM((CHUNK, d), jnp.float32),
        pltpu.VMEM((ROWS_PER_TILE, d), jnp.float32),
        pltpu.VMEM_SHARED((N_NODES, d), jnp.float32),
        pltpu.SemaphoreType.DMA,
        pltpu.SemaphoreType.DMA,
        pltpu.SemaphoreType.DMA,
        pltpu.SemaphoreType.DMA,
    ]
    if stage_g:
        scratch.append(pltpu.VMEM_SHARED((N_NODES, d), jnp.float32))
    return pl.kernel(
        functools.partial(_agg_body, d, stage_g),
        out_type=jax.ShapeDtypeStruct((NC, N_NODES, d), jnp.float32),
        mesh=plsc.VectorSubcoreMesh(**_MESH),
        scratch_types=scratch,
        compiler_params=_SC_PARAMS,
    )(g, src_p, dst_p, ew_p)


_TCR = 1000


def _tc1_body(degp_ref, x_ref, w1t_ref, dis_ref, g_ref):
    degp_t = jnp.transpose(degp_ref[...])
    deg = 1.0 + jnp.sum(degp_t, axis=1, keepdims=True)
    pos = deg > 0
    dis = jnp.where(pos, lax.rsqrt(jnp.where(pos, deg, 1.0)), 0.0)
    h = jnp.dot(x_ref[...], w1t_ref[...], preferred_element_type=jnp.float32)
    dis_ref[...] = dis
    g_ref[...] = dis * h


def _tc2_body(aggp_ref, g1_ref, dis_ref, b1_ref, w2t_ref, g2_ref):
    agg = aggp_ref[0] + aggp_ref[1] + g1_ref[...]
    out1 = jnp.maximum(dis_ref[...] * agg + b1_ref[...], 0.0)
    g2_ref[...] = dis_ref[...] * jnp.dot(
        out1, w2t_ref[...], preferred_element_type=jnp.float32)


def _tc3_body(aggp_ref, g2_ref, dis_ref, b2_ref, out_ref):
    agg = aggp_ref[0] + aggp_ref[1] + g2_ref[...]
    out_ref[...] = dis_ref[...] * agg + b2_ref[...]


@jax.jit
def _tc1_call(degp, x, w1t):
    return pl.pallas_call(
        _tc1_body,
        out_shape=[
            jax.ShapeDtypeStruct((N_NODES, 1), jnp.float32),
            jax.ShapeDtypeStruct((N_NODES, HID_DIM), jnp.float32),
        ],
    )(degp, x, w1t)


@jax.jit
def _tc2_call(aggp1, g1, dis, b1r, w2t):
    return pl.pallas_call(
        _tc2_body,
        out_shape=jax.ShapeDtypeStruct((N_NODES, OUT_DIM), jnp.float32),
    )(aggp1, g1, dis, b1r, w2t)


@jax.jit
def _tc3_call(aggp2, g2, dis, b2r):
    return pl.pallas_call(
        _tc3_body,
        out_shape=jax.ShapeDtypeStruct((N_NODES, OUT_DIM), jnp.float32),
    )(aggp2, g2, dis, b2r)


def kernel(x, edge_index, edge_weight, W1, b1, W2, b2):
    src = edge_index[0]
    dst = edge_index[1]
    shard = (NC * NS, CHUNKS_PER_TILE, CHUNK)
    flat = (NC * NS, EDGES_PER_TILE)
    src_p = src.reshape(shard)
    dst_p = dst.reshape(shard)
    ew_p = edge_weight.reshape(shard)

    degp = _deg_call(dst.reshape(flat), edge_weight.reshape(flat))
    dis, g1 = _tc1_call(degp, x, W1.T)
    aggp1 = _agg_call(HID_DIM, False, g1, src_p, dst_p, ew_p)
    g2 = _tc2_call(aggp1, g1, dis, b1.reshape(1, -1), W2.T)
    aggp2 = _agg_call(OUT_DIM, True, g2, src_p, dst_p, ew_p)
    return _tc3_call(aggp2, g2, dis, b2.reshape(1, -1))


# speedup vs baseline: 2.7613x; 1.2079x over previous
import functools

import jax
import jax.numpy as jnp
from jax import lax
from jax.experimental import pallas as pl
from jax.experimental.pallas import tpu as pltpu
from jax.experimental.pallas import tpu_sc as plsc

N_NODES = 10000
N_EDGES = 320000
IN_DIM = 128
HID_DIM = 64
OUT_DIM = 32

NC = 2
NS = 16
CHUNK = 100
CHUNKS_PER_TILE = 100
EDGES_PER_TILE = CHUNKS_PER_TILE * CHUNK
ROWS_PER_TILE = N_NODES // NS

_MESH = dict(core_axis_name="c", subcore_axis_name="s", num_cores=NC,
             num_subcores=NS)
_SC_PARAMS = pltpu.CompilerParams(needs_layout_passes=False,
                                  use_tc_tiling_on_sc=False)


def _deg_body(dst_hbm, ew_hbm, degp_hbm, dst_v, ew_v, degp_v):
    c = lax.axis_index("c")
    s = lax.axis_index("s")
    wid = c * NS + s
    pltpu.sync_copy(dst_hbm.at[wid], dst_v)
    pltpu.sync_copy(ew_hbm.at[wid], ew_v)

    def zero(i, _):
        degp_v[pl.ds(i * 16, 16)] = jnp.zeros((16,), jnp.float32)
        return 0

    lax.fori_loop(0, N_NODES // 16, zero, 0)

    def chunk(j, _):
        idx = dst_v[pl.ds(j * 16, 16)]
        val = ew_v[pl.ds(j * 16, 16)]
        plsc.addupdate_scatter(degp_v, [idx], val)
        return 0

    lax.fori_loop(0, EDGES_PER_TILE // 16, chunk, 0)
    pltpu.sync_copy(degp_v, degp_hbm.at[wid])


@jax.jit
def _deg_call(dst_p, ew_p):
    return pl.kernel(
        _deg_body,
        out_type=jax.ShapeDtypeStruct((NC * NS, N_NODES), jnp.float32),
        mesh=plsc.VectorSubcoreMesh(**_MESH),
        scratch_types=[
            pltpu.VMEM((EDGES_PER_TILE,), jnp.int32),
            pltpu.VMEM((EDGES_PER_TILE,), jnp.float32),
            pltpu.VMEM((N_NODES,), jnp.float32),
        ],
        compiler_params=_SC_PARAMS,
    )(dst_p, ew_p)


def _agg_body(d, stage_g, g_hbm, src_hbm, dst_hbm, ew_hbm, aggp_hbm,
              src_v, dst_v, ew_v, rows_a, rows_b, rows_c, rows_d, acc_sh,
              gs_a, gs_b, gs_c, gs_d, ss_a, ss_b, ss_c, ss_d, *maybe_g_sh):
    nd = d // 16
    c = lax.axis_index("c")
    s = lax.axis_index("s")
    wid = c * NS + s
    pltpu.sync_copy(src_hbm.at[wid], src_v)
    pltpu.sync_copy(dst_hbm.at[wid], dst_v)
    pltpu.sync_copy(ew_hbm.at[wid], ew_v)
    if stage_g:
        g_src = maybe_g_sh[0]
        row0 = s * ROWS_PER_TILE
        pltpu.sync_copy(g_hbm.at[pl.ds(row0, ROWS_PER_TILE)],
                        g_src.at[pl.ds(row0, ROWS_PER_TILE)])
    else:
        g_src = g_hbm

    def zero(i, _):
        for k in range(nd):
            rows_a[i, pl.ds(k * 16, 16)] = jnp.zeros((16,), jnp.float32)
        return 0

    lax.fori_loop(0, CHUNK, zero, 0)
    row0 = s * ROWS_PER_TILE
    for q in range(ROWS_PER_TILE // CHUNK):
        pltpu.sync_copy(rows_a, acc_sh.at[pl.ds(row0 + q * CHUNK, CHUNK)])
    rem = ROWS_PER_TILE % CHUNK
    if rem:
        pltpu.sync_copy(rows_a.at[pl.ds(0, rem)],
                        acc_sh.at[pl.ds(row0 + ROWS_PER_TILE - rem, rem)])
    plsc.subcore_barrier()

    def scale(j, buf):
        def body(i, _):
            sc = plsc.load_gather(
                ew_v, [jnp.full((16,), j, jnp.int32), jnp.full((16,), i, jnp.int32)])
            for k in range(nd):
                buf[i, pl.ds(k * 16, 16)] = buf[i, pl.ds(k * 16, 16)] * sc
            return 0

        lax.fori_loop(0, CHUNK, body, 0)

    def issue_gather(j, buf, sem):
        pltpu.async_copy(g_src.at[src_v.at[j]], buf, sem)

    def wait_gather(buf, sem):
        pltpu.make_async_copy(g_src.at[src_v.at[0]], buf, sem).wait()

    def issue_scatter(j, buf, sem):
        pltpu.async_copy(buf, acc_sh.at[dst_v.at[j]], sem, add=True)

    def wait_scatter(buf, sem):
        pltpu.make_async_copy(buf, acc_sh.at[dst_v.at[0]], sem).wait()

    bufs = (rows_a, rows_b, rows_c, rows_d)
    gss = (gs_a, gs_b, gs_c, gs_d)
    sss = (ss_a, ss_b, ss_c, ss_d)
    last_j = CHUNKS_PER_TILE - 1

    def step(j, i, do_wait_s, do_issue_g):
        wait_gather(bufs[i], gss[i])
        scale(j, bufs[i])
        issue_scatter(j, bufs[i], sss[i])
        if do_wait_s:
            wait_scatter(bufs[(i + 2) % 4], sss[(i + 2) % 4])
        if do_issue_g:
            issue_gather(j + 2, bufs[(i + 2) % 4], gss[(i + 2) % 4])

    issue_gather(0, rows_a, gs_a)
    issue_gather(1, rows_b, gs_b)
    step(0, 0, False, True)
    step(1, 1, False, True)
    step(2, 2, True, True)
    step(3, 3, True, True)

    def quad(m, _):
        j = 4 * m
        for i in range(4):
            step(j + i, i, True, True)
        return 0

    lax.fori_loop(1, CHUNKS_PER_TILE // 4 - 1, quad, 0)
    base = CHUNKS_PER_TILE - 4
    for i in range(4):
        step(base + i, i, True, base + i + 2 <= last_j)
    wait_scatter(bufs[2], sss[2])
    wait_scatter(bufs[3], sss[3])
    plsc.subcore_barrier()
    pltpu.sync_copy(
        acc_sh.at[pl.ds(s * ROWS_PER_TILE, ROWS_PER_TILE)],
        aggp_hbm.at[c, pl.ds(s * ROWS_PER_TILE, ROWS_PER_TILE)])


@functools.partial(jax.jit, static_argnums=(0, 1))
def _agg_call(d, stage_g, g, src_p, dst_p, ew_p):
    scratch = [
        pltpu.VMEM((CHUNKS_PER_TILE, CHUNK), jnp.int32),
        pltpu.VMEM((CHUNKS_PER_TILE, CHUNK), jnp.int32),
        pltpu.VMEM((CHUNKS_PER_TILE, CHUNK), jnp.float32),
        pltpu.VMEM((CHUNK, d), jnp.float32),
        pltpu.VMEM((CHUNK, d), jnp.float32),
        pltpu.VMEM((CHUNK, d), jnp.float32),
        pltpu.VMEM((CHUNK, d), jnp.float32),
        pltpu.VMEM_SHARED((N_NODES, d), jnp.float32),
        pltpu.SemaphoreType.DMA,
        pltpu.SemaphoreType.DMA,
        pltpu.SemaphoreType.DMA,
        pltpu.SemaphoreType.DMA,
        pltpu.SemaphoreType.DMA,
        pltpu.SemaphoreType.DMA,
        pltpu.SemaphoreType.DMA,
        pltpu.SemaphoreType.DMA,
    ]
    if stage_g:
        scratch.append(pltpu.VMEM_SHARED((N_NODES, d), jnp.float32))
    return pl.kernel(
        functools.partial(_agg_body, d, stage_g),
        out_type=jax.ShapeDtypeStruct((NC, N_NODES, d), jnp.float32),
        mesh=plsc.VectorSubcoreMesh(**_MESH),
        scratch_types=scratch,
        compiler_params=_SC_PARAMS,
    )(g, src_p, dst_p, ew_p)


_TCR = 1000


def _tc1_body(degp_ref, x_ref, w1t_ref, dis_ref, g_ref):
    degp_t = jnp.transpose(degp_ref[...])
    deg = 1.0 + jnp.sum(degp_t, axis=1, keepdims=True)
    pos = deg > 0
    dis = jnp.where(pos, lax.rsqrt(jnp.where(pos, deg, 1.0)), 0.0)
    h = jnp.dot(x_ref[...], w1t_ref[...], preferred_element_type=jnp.float32)
    dis_ref[...] = dis
    g_ref[...] = dis * h


def _tc2_body(aggp_ref, g1_ref, dis_ref, b1_ref, w2t_ref, g2_ref):
    agg = aggp_ref[0] + aggp_ref[1] + g1_ref[...]
    out1 = jnp.maximum(dis_ref[...] * agg + b1_ref[...], 0.0)
    g2_ref[...] = dis_ref[...] * jnp.dot(
        out1, w2t_ref[...], preferred_element_type=jnp.float32)


def _tc3_body(aggp_ref, g2_ref, dis_ref, b2_ref, out_ref):
    agg = aggp_ref[0] + aggp_ref[1] + g2_ref[...]
    out_ref[...] = dis_ref[...] * agg + b2_ref[...]


@jax.jit
def _tc1_call(degp, x, w1t):
    return pl.pallas_call(
        _tc1_body,
        out_shape=[
            jax.ShapeDtypeStruct((N_NODES, 1), jnp.float32),
            jax.ShapeDtypeStruct((N_NODES, HID_DIM), jnp.float32),
        ],
    )(degp, x, w1t)


@jax.jit
def _tc2_call(aggp1, g1, dis, b1r, w2t):
    return pl.pallas_call(
        _tc2_body,
        out_shape=jax.ShapeDtypeStruct((N_NODES, OUT_DIM), jnp.float32),
    )(aggp1, g1, dis, b1r, w2t)


@jax.jit
def _tc3_call(aggp2, g2, dis, b2r):
    return pl.pallas_call(
        _tc3_body,
        out_shape=jax.ShapeDtypeStruct((N_NODES, OUT_DIM), jnp.float32),
    )(aggp2, g2, dis, b2r)


def kernel(x, edge_index, edge_weight, W1, b1, W2, b2):
    src = edge_index[0]
    dst = edge_index[1]
    shard = (NC * NS, CHUNKS_PER_TILE, CHUNK)
    flat = (NC * NS, EDGES_PER_TILE)
    src_p = src.reshape(shard)
    dst_p = dst.reshape(shard)
    ew_p = edge_weight.reshape(shard)

    degp = _deg_call(dst.reshape(flat), edge_weight.reshape(flat))
    dis, g1 = _tc1_call(degp, x, W1.T)
    aggp1 = _agg_call(HID_DIM, False, g1, src_p, dst_p, ew_p)
    g2 = _tc2_call(aggp1, g1, dis, b1.reshape(1, -1), W2.T)
    aggp2 = _agg_call(OUT_DIM, True, g2, src_p, dst_p, ew_p)
    return _tc3_call(aggp2, g2, dis, b2.reshape(1, -1))
